# Initial kernel scaffold; baseline (speedup 1.0000x reference)
import jax
import jax.numpy as jnp
from jax.experimental import pallas as pl


def kernel(x, edge_index, W1, b1, Wmu, bmu, Wls, bls):
    raise NotImplementedError("write your pallas kernel here")


# speedup vs baseline: 21.8363x; 21.8363x over previous
import functools

import jax
import jax.numpy as jnp
from jax import lax
from jax.experimental import pallas as pl
from jax.experimental.pallas import tpu as pltpu
from jax.experimental.pallas import tpu_sc as plsc

NC = 2
NS = 16
LANES = 128


def _round_up(a, b):
    return (a + b - 1) // b * b


def _sc_deg(dst3, np_, cpw):
    rps = np_ // NS

    mesh = plsc.VectorSubcoreMesh(core_axis_name="c", subcore_axis_name="s")

    @functools.partial(
        pl.kernel,
        out_type=jax.ShapeDtypeStruct((NC * np_,), jnp.float32),
        mesh=mesh,
        scratch_types=[
            pltpu.VMEM((cpw, LANES), jnp.int32),
            pltpu.VMEM((LANES,), jnp.float32),
            pltpu.VMEM((rps,), jnp.float32),
            pltpu.VMEM_SHARED((np_,), jnp.float32),
        ],
    )
    def deg_kernel(dst_hbm, degp_hbm, dstv, onesv, zrow, degsh):
        core = lax.axis_index("c")
        sub = lax.axis_index("s")
        wid = sub * NC + core
        pltpu.sync_copy(dst_hbm.at[wid], dstv)

        for k in range(LANES // 16):
            onesv[pl.ds(k * 16, 16)] = jnp.full((16,), 1.0, jnp.float32)

        def zbody(i, _):
            zrow[pl.ds(pl.multiple_of(i * 16, 16), 16)] = jnp.zeros(
                (16,), jnp.float32)
            return 0

        lax.fori_loop(0, rps // 16, zbody, 0)

        base = pl.multiple_of(sub * rps, 128)
        pltpu.sync_copy(zrow, degsh.at[pl.ds(base, rps)])
        plsc.subcore_barrier()

        def ebody(j, _):
            pltpu.sync_copy(onesv, degsh.at[dstv.at[j]], add=True)
            return 0

        lax.fori_loop(0, cpw, ebody, 0)
        plsc.subcore_barrier()

        obase = pl.multiple_of(core * np_ + sub * rps, 128)
        pltpu.sync_copy(degsh.at[pl.ds(base, rps)], degp_hbm.at[pl.ds(obase, rps)])

    return deg_kernel(dst3)


def _sc_edges(src3, dst3, xs, dis, np_, cpw, fin):
    rps = np_ // NS

    mesh = plsc.VectorSubcoreMesh(core_axis_name="c", subcore_axis_name="s")

    @functools.partial(
        pl.kernel,
        out_type=(
            jax.ShapeDtypeStruct((NC * np_, fin), jnp.float32),
            jax.ShapeDtypeStruct((NC * np_,), jnp.float32),
        ),
        mesh=mesh,
        scratch_types=[
            pltpu.VMEM((cpw, LANES), jnp.int32),
            pltpu.VMEM((cpw, LANES), jnp.int32),
            pltpu.VMEM((LANES, fin), jnp.float32),
            pltpu.VMEM((LANES,), jnp.float32),
            pltpu.VMEM((rps,), jnp.float32),
            pltpu.VMEM_SHARED((np_, fin), jnp.float32),
            pltpu.VMEM_SHARED((np_,), jnp.float32),
        ],
    )
    def edge_kernel(src_hbm, dst_hbm, xs_hbm, dis_hbm, tp_hbm, cp_hbm,
                    srcv, dstv, rows, dvals, zrow, tsh, csh):
        core = lax.axis_index("c")
        sub = lax.axis_index("s")
        wid = sub * NC + core
        pltpu.sync_copy(src_hbm.at[wid], srcv)
        pltpu.sync_copy(dst_hbm.at[wid], dstv)

        def zr(i, _):
            for k in range(fin // 16):
                rows[i, pl.ds(k * 16, 16)] = jnp.zeros((16,), jnp.float32)
            return 0

        lax.fori_loop(0, LANES, zr, 0)

        def zc(i, _):
            zrow[pl.ds(pl.multiple_of(i * 16, 16), 16)] = jnp.zeros(
                (16,), jnp.float32)
            return 0

        lax.fori_loop(0, rps // 16, zc, 0)

        base = pl.multiple_of(sub * rps, 128)
        for k in range(rps // LANES):
            pltpu.sync_copy(rows, tsh.at[pl.ds(base + k * LANES, LANES)])
        pltpu.sync_copy(zrow, csh.at[pl.ds(base, rps)])
        plsc.subcore_barrier()

        def ebody(j, _):
            si = srcv.at[j]
            di = dstv.at[j]
            pltpu.sync_copy(xs_hbm.at[si], rows)
            pltpu.sync_copy(rows, tsh.at[di], add=True)
            pltpu.sync_copy(dis_hbm.at[di], dvals)
            pltpu.sync_copy(dvals, csh.at[si], add=True)
            return 0

        lax.fori_loop(0, cpw, ebody, 0)
        plsc.subcore_barrier()

        obase = pl.multiple_of(core * np_ + sub * rps, 128)
        for k in range(rps // LANES):
            pltpu.sync_copy(tsh.at[pl.ds(base + k * LANES, LANES)],
                            tp_hbm.at[pl.ds(obase + k * LANES, LANES)])
        pltpu.sync_copy(csh.at[pl.ds(base, rps)], cp_hbm.at[pl.ds(obase, rps)])

    return edge_kernel(src3, dst3, xs, dis)


def _tc_matmul(x_pad, w1, np_, fin, bs):
    def body(x_ref, w_ref, o_ref):
        o_ref[:] = jnp.dot(x_ref[:], w_ref[:],
                           preferred_element_type=jnp.float32)

    return pl.pallas_call(
        body,
        grid=(np_ // bs,),
        in_specs=[
            pl.BlockSpec((bs, fin), lambda i: (i, 0)),
            pl.BlockSpec((fin, fin), lambda i: (0, 0)),
        ],
        out_specs=pl.BlockSpec((bs, fin), lambda i: (i, 0)),
        out_shape=jax.ShapeDtypeStruct((np_, fin), jnp.float32),
    )(x_pad, w1)


def _tc_prep(xw, degp, n, np_, fin, bs):
    def body(xw_ref, degp_ref, xs_ref, dis_ref):
        i = pl.program_id(0)
        dp = degp_ref[:]
        degsum = 1.0 + dp[0] + dp[1]
        rows = lax.broadcasted_iota(jnp.int32, (bs, 1), 0) + i * bs
        dis = jnp.where(rows < n, lax.rsqrt(degsum), 0.0)
        xs_ref[:] = xw_ref[:] * dis
        dis_ref[:] = dis

    return pl.pallas_call(
        body,
        grid=(np_ // bs,),
        in_specs=[
            pl.BlockSpec((bs, fin), lambda i: (i, 0)),
            pl.BlockSpec((NC, bs, 1), lambda i: (0, i, 0)),
        ],
        out_specs=[
            pl.BlockSpec((bs, fin), lambda i: (i, 0)),
            pl.BlockSpec((bs, 1), lambda i: (i, 0)),
        ],
        out_shape=(
            jax.ShapeDtypeStruct((np_, fin), jnp.float32),
            jax.ShapeDtypeStruct((np_, 1), jnp.float32),
        ),
    )(xw, degp)


def _tc_gsum(tp, xs, dis, cp, b1, np_, fin, bs):
    nblk = np_ // bs

    def body(tp_ref, xs_ref, dis_ref, cp_ref, b1_ref, g_ref):
        tp2 = tp_ref[:]
        t = tp2[0] + tp2[1]
        dis = dis_ref[:]
        cp2 = cp_ref[:]
        h = jnp.maximum(dis * (t + xs_ref[:]) + b1_ref[:], 0.0)
        wv = dis * (cp2[0] + cp2[1] + dis)
        g = lax.dot_general(wv, h, (((0,), (0,)), ((), ())),
                            preferred_element_type=jnp.float32)
        g_ref[:] = g.reshape(g_ref.shape)

    return pl.pallas_call(
        body,
        grid=(nblk,),
        in_specs=[
            pl.BlockSpec((NC, bs, fin), lambda i: (0, i, 0)),
            pl.BlockSpec((bs, fin), lambda i: (i, 0)),
            pl.BlockSpec((bs, 1), lambda i: (i, 0)),
            pl.BlockSpec((NC, bs, 1), lambda i: (0, i, 0)),
            pl.BlockSpec((1, fin), lambda i: (0, 0)),
        ],
        out_specs=pl.BlockSpec((1, 1, fin), lambda i: (i, 0, 0)),
        out_shape=jax.ShapeDtypeStruct((nblk, 1, fin), jnp.float32),
    )(tp, xs, dis, cp, b1)


def _tc_heads(gparts, wmu, bmu, wls, bls, n, fin, fout, bs):
    nblk = n // bs
    inv_n = 1.0 / n

    def body(g_ref, wmu_ref, bmu_ref, wls_ref, bls_ref, omu_ref, ols_ref):
        g = jnp.sum(g_ref[:], axis=0, keepdims=True) * inv_n
        mu = jnp.dot(g, wmu_ref[:], preferred_element_type=jnp.float32) \
            + bmu_ref[:]
        ls = jnp.dot(g, wls_ref[:], preferred_element_type=jnp.float32) \
            + bls_ref[:]
        omu_ref[:] = jnp.broadcast_to(mu, (bs, mu.shape[1]))
        ols_ref[:] = jnp.broadcast_to(ls, (bs, ls.shape[1]))

    nparts = gparts.shape[0]
    return pl.pallas_call(
        body,
        grid=(nblk,),
        in_specs=[
            pl.BlockSpec((nparts, fin), lambda i: (0, 0)),
            pl.BlockSpec((fin, fout), lambda i: (0, 0)),
            pl.BlockSpec((1, fout), lambda i: (0, 0)),
            pl.BlockSpec((fin, fout), lambda i: (0, 0)),
            pl.BlockSpec((1, fout), lambda i: (0, 0)),
        ],
        out_specs=[
            pl.BlockSpec((bs, fout), lambda i: (i, 0)),
            pl.BlockSpec((bs, fout), lambda i: (i, 0)),
        ],
        out_shape=(
            jax.ShapeDtypeStruct((n, fout), jnp.float32),
            jax.ShapeDtypeStruct((n, fout), jnp.float32),
        ),
    )(gparts, wmu, bmu, wls, bls)


def kernel(x, edge_index, W1, b1, Wmu, bmu, Wls, bls):
    n, fin = x.shape
    e = edge_index.shape[1]
    fout = Wmu.shape[1]
    nw = NC * NS

    np_ = _round_up(n + 1, NS * LANES)
    ep = _round_up(e, nw * LANES)
    cpw = ep // (nw * LANES)

    src = edge_index[0]
    dst = edge_index[1]
    pad_e = ep - e
    src_p = jnp.concatenate(
        [src, jnp.zeros((pad_e,), jnp.int32)]).reshape(nw, cpw, LANES)
    dst_p = jnp.concatenate(
        [dst, jnp.full((pad_e,), n, jnp.int32)]).reshape(nw, cpw, LANES)

    x_pad = jnp.pad(x, ((0, np_ - n), (0, 0)))

    degp = _sc_deg(dst_p, np_, cpw)
    xw = _tc_matmul(x_pad, W1, np_, fin, 1024)

    degp2 = degp.reshape(NC, np_, 1)
    xs, dis2 = _tc_prep(xw, degp2, n, np_, fin, 1024)

    tp, cp = _sc_edges(src_p, dst_p, xs, dis2.reshape(np_), np_, cpw, fin)

    gparts = _tc_gsum(tp.reshape(NC, np_, fin), xs, dis2,
                      cp.reshape(NC, np_, 1), b1.reshape(1, fin),
                      np_, fin, 1024).reshape(-1, fin)

    out_mu, out_ls = _tc_heads(gparts, Wmu, bmu.reshape(1, fout),
                               Wls, bls.reshape(1, fout), n, fin, fout, 1000)
    return (out_mu, out_ls)
